# Initial kernel scaffold; baseline (speedup 1.0000x reference)
import jax
import jax.numpy as jnp
from jax.experimental import pallas as pl


def kernel(expressions_encodings, symbols_encodings, expr_idx, token_idx, symbol_idx, Wu, bu, Wg, bg):
    raise NotImplementedError("write your pallas kernel here")


# speedup vs baseline: 1.0355x; 1.0355x over previous
import functools

import jax
import jax.numpy as jnp
from jax.experimental import pallas as pl
from jax.experimental.pallas import tpu as pltpu

_BLK = 2000


def _gate_body(occ_ref, sym_ref, wu_ref, bu_ref, wg1_ref, wg2_ref, bg_ref, out_ref):
    occ = occ_ref[...]
    sym = sym_ref[...]
    u = jnp.dot(sym, wu_ref[...], preferred_element_type=jnp.float32) + bu_ref[...]
    u = jnp.maximum(u, 0.0)
    z = (jnp.dot(occ, wg1_ref[...], preferred_element_type=jnp.float32)
         + jnp.dot(u, wg2_ref[...], preferred_element_type=jnp.float32)
         + bg_ref[...])
    g = jax.nn.sigmoid(z)
    out_ref[...] = g * occ + (1.0 - g) * u


def _gate(occ, sym, Wu, bu, Wg1, Wg2, bg):
    e, d = occ.shape
    e_pad = ((e + _BLK - 1) // _BLK) * _BLK
    if e_pad != e:
        occ = jnp.pad(occ, ((0, e_pad - e), (0, 0)))
        sym = jnp.pad(sym, ((0, e_pad - e), (0, 0)))
    grid = e_pad // _BLK
    row_spec = pl.BlockSpec((_BLK, d), lambda i: (i, 0))
    full_spec = pl.BlockSpec((d, d), lambda i: (0, 0))
    bias_spec = pl.BlockSpec((1, d), lambda i: (0, 0))
    out = pl.pallas_call(
        _gate_body,
        grid=(grid,),
        in_specs=[row_spec, row_spec, full_spec, bias_spec, full_spec,
                  full_spec, bias_spec],
        out_specs=row_spec,
        out_shape=jax.ShapeDtypeStruct((e_pad, d), jnp.float32),
    )(occ, sym, Wu, bu.reshape(1, d), Wg1, Wg2, bg.reshape(1, d))
    return out[:e]


def kernel(expressions_encodings, symbols_encodings, expr_idx, token_idx,
           symbol_idx, Wu, bu, Wg, bg):
    b, t, d = expressions_encodings.shape
    flat = expressions_encodings.reshape(b * t, d)
    flat_idx = t * expr_idx + token_idx
    occ = jnp.take(flat, flat_idx, axis=0)
    sym = jnp.take(symbols_encodings, symbol_idx, axis=0)
    upd = _gate(occ, sym, Wu, bu, Wg[:d], Wg[d:], bg)
    out = flat.at[flat_idx].set(upd)
    return out.reshape(b, t, d)
